# baseline (speedup 1.0000x reference)
import functools
import jax
import jax.numpy as jnp
from jax import lax
from jax.experimental import pallas as pl
from jax.experimental.pallas import tpu as pltpu

N = 4096
NFEAT = 128
NHID = 64
NCLASS = 16
BM = 256
NBLK = N // BM
NADJ = 5
F32 = jnp.float32


def _body(x_ref, a0, a1, a2, a3, a4,
          w1c, b1c, w2s, b2s,
          aw1, ab1, aw2, gw1, gb1, gw2, mw, mb,
          out_ref, fadj_ref, st_ref, sem_ref, emb_ref,
          U, V, FS):
    p = pl.program_id(0)
    i = pl.program_id(1)
    a = pl.program_id(2)
    adj_refs = (a0, a1, a2, a3, a4)

    @pl.when(jnp.logical_and(p == 0, jnp.logical_and(i == 0, a == 0)))
    def _():
        U[...] = jnp.dot(x_ref[...], w1c[...], preferred_element_type=F32)

    @pl.when(p == 0)
    def _():
        def mk1(g):
            def f():
                A = adj_refs[g][...]
                m = jnp.dot(A, U[:, g * NHID:(g + 1) * NHID],
                            preferred_element_type=F32)
                h = jnp.maximum(m + b1c[0, g * NHID:(g + 1) * NHID], 0.0)
                V[g, pl.ds(i * BM, BM), :] = jnp.dot(
                    h, w2s[g], preferred_element_type=F32)
            return f
        lax.switch(a, [mk1(g) for g in range(NADJ)])

    @pl.when(p == 1)
    def _():
        def attend(zs, W1, b1, W2):
            ws = []
            for z in zs:
                t = jnp.tanh(jnp.dot(z, W1, preferred_element_type=F32)
                             + b1[0])
                ws.append(jnp.sum(t * W2[0], axis=1, keepdims=True))
            m = functools.reduce(jnp.maximum, ws)
            es = [jnp.exp(w - m) for w in ws]
            denom = functools.reduce(jnp.add, es)
            out = jnp.zeros_like(zs[0])
            for z, e in zip(zs, es):
                out = out + (e / denom) * z
            return out

        def mk2(g):
            def f():
                A = adj_refs[g][...]
                o = jnp.dot(A, V[g], preferred_element_type=F32) + b2s[g]
                if g < 3:
                    FS[g] = o
                elif g == 3:
                    st_ref[...] = o
                    FS[3] = o
                else:
                    sem_ref[...] = o
                    f1, f2, f3, stv = FS[0], FS[1], FS[2], FS[3]
                    fadj = attend([f1, f2, f3], aw1[...], ab1, aw2)
                    fadj_ref[...] = fadj
                    emb = attend([fadj, stv, o], gw1[...], gb1, gw2)
                    emb_ref[...] = emb
                    logits = (jnp.dot(emb, mw[...],
                                      preferred_element_type=F32) + mb[0])
                    mx = jnp.max(logits, axis=1, keepdims=True)
                    lse = mx + jnp.log(
                        jnp.sum(jnp.exp(logits - mx), axis=1, keepdims=True))
                    out_ref[...] = logits - lse
            return f
        lax.switch(a, [mk2(g) for g in range(NADJ)])


def _adj_spec(g):
    def imap(p, i, a):
        row = jnp.where(a < g, i - 1, i)
        return (jnp.maximum(row, 0), 0)
    return pl.BlockSpec((BM, N), imap)


def _const_spec(shape):
    nd = len(shape)
    return pl.BlockSpec(shape, lambda p, i, a: (0,) * nd)


def _out_spec(cols):
    return pl.BlockSpec((BM, cols),
                        lambda p, i, a: (jnp.where(p == 1, i, 0), 0))


@jax.jit
def kernel(x, sadj, fadj1, fadj2, fadj3, ppmi, params):
    w1c = jnp.concatenate(
        [params[nm]["W1"] for nm in ("F1", "F2", "F3", "SGCN", "SEM")], axis=1)
    b1c = jnp.concatenate(
        [params[nm]["b1"] for nm in ("F1", "F2", "F3", "SGCN", "SEM")]
    ).reshape(1, NADJ * NHID)
    w2s = jnp.stack(
        [params[nm]["W2"] for nm in ("F1", "F2", "F3", "SGCN", "SEM")])
    b2s = jnp.stack(
        [params[nm]["b2"] for nm in ("F1", "F2", "F3", "SGCN", "SEM")])
    att, aall, mlp = params["att"], params["att_all"], params["mlp"]

    grid = (2, NBLK, NADJ)
    out_shapes = (
        jax.ShapeDtypeStruct((N, NCLASS), F32),
        jax.ShapeDtypeStruct((N, NHID), F32),
        jax.ShapeDtypeStruct((N, NHID), F32),
        jax.ShapeDtypeStruct((N, NHID), F32),
        jax.ShapeDtypeStruct((N, NHID), F32),
    )
    res = pl.pallas_call(
        _body,
        grid=grid,
        in_specs=[
            _const_spec((N, NFEAT)),
            _adj_spec(0), _adj_spec(1), _adj_spec(2),
            _adj_spec(3), _adj_spec(4),
            _const_spec((NFEAT, NADJ * NHID)),
            _const_spec((1, NADJ * NHID)),
            _const_spec((NADJ, NHID, NHID)),
            _const_spec((NADJ, NHID)),
            _const_spec((NHID, 16)),
            _const_spec((1, 16)),
            _const_spec((1, 16)),
            _const_spec((NHID, 32)),
            _const_spec((1, 32)),
            _const_spec((1, 32)),
            _const_spec((NHID, NCLASS)),
            _const_spec((1, NCLASS)),
        ],
        out_specs=(
            _out_spec(NCLASS), _out_spec(NHID), _out_spec(NHID),
            _out_spec(NHID), _out_spec(NHID),
        ),
        out_shape=out_shapes,
        scratch_shapes=[
            pltpu.VMEM((N, NADJ * NHID), F32),
            pltpu.VMEM((NADJ, N, NHID), F32),
            pltpu.VMEM((4, BM, NHID), F32),
        ],
        compiler_params=pltpu.CompilerParams(
            dimension_semantics=("arbitrary", "arbitrary", "arbitrary"),
            vmem_limit_bytes=64 * 1024 * 1024,
        ),
    )(
        x, fadj1, fadj2, fadj3, sadj, ppmi,
        w1c, b1c, w2s, b2s,
        att["W1"], att["b1"].reshape(1, 16), att["W2"].reshape(1, 16),
        aall["W1"], aall["b1"].reshape(1, 32), aall["W2"].reshape(1, 32),
        mlp["W"], mlp["b"].reshape(1, NCLASS),
    )
    return res
